# baseline (speedup 1.0000x reference)
import jax
import jax.numpy as jnp
from jax.experimental import pallas as pl
from jax.experimental.pallas import tpu as pltpu


def _pool4d_kernel(x_ref, o_ref):
    x = x_ref[...]
    m = jnp.max(x, axis=3)
    o_ref[...] = jnp.max(m, axis=2)

def kernel(x):
    N, C, H, W = x.shape
    BN, BC = 8, 128
    grid = (N // BN, C // BC)
    out2d = pl.pallas_call(
        _pool4d_kernel,
        out_shape=jax.ShapeDtypeStruct((N, C), x.dtype),
        grid=grid,
        in_specs=[pl.BlockSpec((BN, BC, H, W), lambda i, j: (i, j, 0, 0))],
        out_specs=pl.BlockSpec((BN, BC), lambda i, j: (i, j)),
        compiler_params=pltpu.CompilerParams(
            dimension_semantics=("parallel", "arbitrary"),
            vmem_limit_bytes=64 * 1024 * 1024,
        ),
    )(x)
    return out2d.reshape(N, C, 1, 1)


# speedup vs baseline: 23.9918x; 23.9918x over previous
import jax
import jax.numpy as jnp
from jax.experimental import pallas as pl
from jax.experimental.pallas import tpu as pltpu


def _plane_max_kernel(x_ref, o_ref):
    o_ref[...] = jnp.max(x_ref[...], axis=(0, 1))


def kernel(x):
    N, C, H, W = x.shape
    xt = jnp.transpose(x, (2, 3, 0, 1))

    BN = 16
    grid = (N // BN,)
    out2d = pl.pallas_call(
        _plane_max_kernel,
        out_shape=jax.ShapeDtypeStruct((N, C), x.dtype),
        grid=grid,
        in_specs=[pl.BlockSpec((H, W, BN, C), lambda i: (0, 0, i, 0))],
        out_specs=pl.BlockSpec((BN, C), lambda i: (i, 0)),
        compiler_params=pltpu.CompilerParams(
            dimension_semantics=("parallel",),
            vmem_limit_bytes=64 * 1024 * 1024,
        ),
    )(xt)
    return out2d.reshape(N, C, 1, 1)
